# Initial kernel scaffold; baseline (speedup 1.0000x reference)
import jax
import jax.numpy as jnp
from jax.experimental import pallas as pl


def kernel(x0, w0):
    raise NotImplementedError("write your pallas kernel here")


# speedup vs baseline: 1.0046x; 1.0046x over previous
import functools

import numpy as np
import jax
import jax.numpy as jnp
from jax import lax
from jax.experimental import pallas as pl
from jax.experimental.pallas import tpu as pltpu
from jax.experimental.pallas import tpu_sc as plsc

_N = 65536
_D = 32
_NT = 3
_NK = 10

_MUS = np.linspace(-2.0, 2.0, _NT)
_SIGMAS = np.linspace(3.0, 1.0, _NT)

_M32 = 0xFFFFFFFF


def _tf2x32(k1, k2, c1, c2):
    ks0, ks1 = k1, k2
    ks2 = ks0 ^ ks1 ^ 0x1BD11BDA

    def rot(v, d):
        return ((v << d) | (v >> (32 - d))) & _M32

    def rounds(x0, x1, rs):
        for r in rs:
            x0 = (x0 + x1) & _M32
            x1 = rot(x1, r)
            x1 = x0 ^ x1
        return x0, x1

    x0 = (c1 + ks0) & _M32
    x1 = (c2 + ks1) & _M32
    x0, x1 = rounds(x0, x1, (13, 15, 26, 6))
    x0, x1 = (x0 + ks1) & _M32, (x1 + ks2 + 1) & _M32
    x0, x1 = rounds(x0, x1, (17, 29, 16, 24))
    x0, x1 = (x0 + ks2) & _M32, (x1 + ks0 + 2) & _M32
    x0, x1 = rounds(x0, x1, (13, 15, 26, 6))
    x0, x1 = (x0 + ks0) & _M32, (x1 + ks1 + 3) & _M32
    x0, x1 = rounds(x0, x1, (17, 29, 16, 24))
    x0, x1 = (x0 + ks1) & _M32, (x1 + ks2 + 4) & _M32
    x0, x1 = rounds(x0, x1, (13, 15, 26, 6))
    x0, x1 = (x0 + ks2) & _M32, (x1 + ks0 + 5) & _M32
    return x0, x1


def _split2(k):
    return _tf2x32(k[0], k[1], 0, 0), _tf2x32(k[0], k[1], 0, 1)


def _key_schedule():
    key = (0, 42)
    sched = []
    for _t in range(_NT - 1):
        kns, kus = [], []
        for _k in range(_NK):
            key, sk = _split2(key)
            kn, ku = _split2(sk)
            kns.append(kn)
            kus.append(ku)
        key, rk = _split2(key)
        sched.append((kns, kus, rk))
    return sched


_KEY_SCHED = _key_schedule()


def _mala_body(consts, x_ref, n_ref, u_ref, o_ref):
    x = x_ref[:, :_D]
    for k in range(_NK):
        tau, s2t, q2t, cq, ds, mu, sig2 = consts[k]
        noise = n_ref[k]
        u = u_ref[k][:, None]
        grad = np.float32(2.0) * (ds * (x - mu))
        new_x = x + tau * grad + s2t * noise

        def ld(y):
            dy = y - mu
            return (np.float32(-0.5) * jnp.sum(dy * dy, axis=-1, keepdims=True)) / sig2

        def log_q(xp, xx):
            mean = xx + tau * (np.float32(2.0) * (ds * (xx - mu)))
            dd = xp - mean
            return (np.float32(-0.5) * jnp.sum(dd * dd, axis=-1, keepdims=True)) / q2t - cq

        alp = ld(new_x) - ld(x) + log_q(x, new_x) - log_q(new_x, x)
        mask = (u < jnp.exp(alp)).astype(jnp.float32)
        x = mask * new_x + (np.float32(1.0) - mask) * x
    o_ref[:, :_D] = x


_MALA_BR = 2048
_W = 128


def _mala(x, noise, us, consts):
    grid = (_N // _MALA_BR,)
    win = x.shape[1]
    return pl.pallas_call(
        functools.partial(_mala_body, consts),
        grid=grid,
        in_specs=[
            pl.BlockSpec((_MALA_BR, win), lambda i: (i, 0)),
            pl.BlockSpec((_NK, _MALA_BR, _D), lambda i: (0, i, 0)),
            pl.BlockSpec((_NK, _MALA_BR), lambda i: (0, i)),
        ],
        out_specs=pl.BlockSpec((_MALA_BR, _W), lambda i: (i, 0)),
        out_shape=jax.ShapeDtypeStruct((_N, _W), jnp.float32),
    )(x, noise, us)


def _mala_consts(t):
    mu = np.float32(float(_MUS[t]))
    sig2 = np.float32(float(_SIGMAS[t]) ** 2)
    ds = np.float32(-0.5) * (np.float32(1.0) / sig2)
    out = []
    for k in range(1, _NK + 1):
        tau = 1.0 / k
        s2t = np.float32(float(jnp.sqrt(jnp.asarray(2.0 * tau, jnp.float32))))
        cq = np.float32(float(0.5 * _D * jnp.log(2.0 * jnp.pi * 2.0 * tau)))
        out.append((np.float32(tau), s2t, np.float32(2.0 * tau), cq, ds, mu, sig2))
    return tuple(out)


_MALA_CONSTS = (_mala_consts(0), _mala_consts(1))
_LOGN = np.float32(float(jnp.log(jnp.asarray(float(_N), jnp.float32))))
_TINY = np.float32(np.finfo(np.float32).tiny)
_MV = np.float32(np.float32(1.0) - _TINY)


_CAT_BR = 256
_CAT_BC = 2048


def _cat_body(rk, br, bc, ncols, l_ref, o_ref, bv_ref, bi_ref):
    i = pl.program_id(0)
    j = pl.program_id(1)
    nj = pl.num_programs(1)
    u32 = np.uint32

    rows = (lax.broadcasted_iota(jnp.int32, (br, bc), 0) + i * br).astype(jnp.uint32)
    cols = (lax.broadcasted_iota(jnp.int32, (br, bc), 1) + j * bc).astype(jnp.uint32)
    lo = rows * u32(ncols) + cols

    ks0, ks1 = rk
    ks2 = ks0 ^ ks1 ^ 0x1BD11BDA

    def rot(v, d):
        return (v << u32(d)) | (v >> u32(32 - d))

    def rounds(x0, x1, rs):
        for r in rs:
            x0 = x0 + x1
            x1 = rot(x1, r)
            x1 = x0 ^ x1
        return x0, x1

    x0 = jnp.full((br, bc), u32(ks0), jnp.uint32)
    x1 = lo + u32(ks1)
    x0, x1 = rounds(x0, x1, (13, 15, 26, 6))
    x0, x1 = x0 + u32(ks1), x1 + u32((ks2 + 1) & _M32)
    x0, x1 = rounds(x0, x1, (17, 29, 16, 24))
    x0, x1 = x0 + u32(ks2), x1 + u32((ks0 + 2) & _M32)
    x0, x1 = rounds(x0, x1, (13, 15, 26, 6))
    x0, x1 = x0 + u32(ks0), x1 + u32((ks1 + 3) & _M32)
    x0, x1 = rounds(x0, x1, (17, 29, 16, 24))
    x0, x1 = x0 + u32(ks1), x1 + u32((ks2 + 4) & _M32)
    x0, x1 = rounds(x0, x1, (13, 15, 26, 6))
    bits = (x0 + u32(ks2)) ^ (x1 + u32((ks0 + 5) & _M32))

    fb = (bits >> u32(9)) | u32(0x3F800000)
    f = lax.bitcast_convert_type(fb, jnp.float32) - np.float32(1.0)
    uu = jnp.maximum(_TINY, f * _MV + _TINY)
    g = -jnp.log(-jnp.log(uu))
    v = g + l_ref[...].reshape(1, bc)

    icols = lax.broadcasted_iota(jnp.int32, (br, bc), 1) + j * bc
    bmax = jnp.max(v, axis=1, keepdims=True)
    bidx = jnp.min(jnp.where(v == bmax, icols, jnp.int32(2147483647)),
                   axis=1, keepdims=True)

    @pl.when(j == 0)
    def _():
        bv_ref[...] = bmax
        bi_ref[...] = bidx

    @pl.when(j != 0)
    def _():
        better = bmax > bv_ref[...]
        bv_ref[...] = jnp.where(better, bmax, bv_ref[...])
        bi_ref[...] = jnp.where(better, bidx, bi_ref[...])

    @pl.when(j == nj - 1)
    def _():
        o_ref[...] = bi_ref[...].reshape(1, br, 1)


def _categorical(rk, logits):
    ni = _N // _CAT_BR
    nj = _N // _CAT_BC
    out = pl.pallas_call(
        functools.partial(_cat_body, rk, _CAT_BR, _CAT_BC, _N),
        grid=(ni, nj),
        in_specs=[
            pl.BlockSpec((1, 1, _CAT_BC), lambda i, j: (j, 0, 0)),
        ],
        out_specs=pl.BlockSpec((1, _CAT_BR, 1), lambda i, j: (i, 0, 0)),
        out_shape=jax.ShapeDtypeStruct((ni, _CAT_BR, 1), jnp.int32),
        scratch_shapes=[
            pltpu.VMEM((_CAT_BR, 1), jnp.float32),
            pltpu.VMEM((_CAT_BR, 1), jnp.int32),
        ],
    )(logits.reshape(nj, 1, _CAT_BC))
    return out.reshape(_N)


def _sc_gather(table, idx):
    info = plsc.get_sparse_core_info()
    nw = info.num_cores * info.num_subcores
    bpw = _N // nw
    chunk = 512
    mesh = plsc.VectorSubcoreMesh(core_axis_name="c", subcore_axis_name="s")

    @functools.partial(
        pl.kernel,
        mesh=mesh,
        out_type=jax.ShapeDtypeStruct((_N, _W), jnp.float32),
        scratch_types=[
            pltpu.VMEM((chunk,), jnp.int32),
            pltpu.VMEM((chunk, _W), jnp.float32),
            pltpu.SemaphoreType.DMA,
        ],
    )
    def gk(table_hbm, idx_hbm, out_hbm, idx_v, rows_v, sem):
        wid = lax.axis_index("s") * info.num_cores + lax.axis_index("c")
        base = wid * bpw
        for c in range(bpw // chunk):
            off = base + c * chunk
            pltpu.sync_copy(idx_hbm.at[pl.ds(off, chunk)], idx_v)
            pltpu.async_copy(table_hbm.at[idx_v], rows_v, sem).wait()
            pltpu.sync_copy(rows_v, out_hbm.at[pl.ds(off, chunk)])

    return gk(table, idx)


def _log_body(w_ref, o_ref):
    o_ref[...] = jnp.log(w_ref[...])


def _log_w(w):
    out = pl.pallas_call(
        _log_body,
        out_shape=jax.ShapeDtypeStruct((512, 128), jnp.float32),
    )(w.reshape(512, 128))
    return out.reshape(_N)


def _ulw_body(consts, x_ref, o_ref):
    mu0, sig20, mu1, sig21, logn = consts
    x = x_ref[:, :_D]

    def ld(y, mu, sig2):
        dy = y - mu
        return (np.float32(-0.5) * jnp.sum(dy * dy, axis=-1, keepdims=True)) / sig2

    o_ref[...] = ld(x, mu1, sig21) - logn - ld(x, mu0, sig20)


def _ulw(x, consts):
    grid = (_N // _MALA_BR,)
    out = pl.pallas_call(
        functools.partial(_ulw_body, consts),
        grid=grid,
        in_specs=[pl.BlockSpec((_MALA_BR, _W), lambda i: (i, 0))],
        out_specs=pl.BlockSpec((_MALA_BR, 1), lambda i: (i, 0)),
        out_shape=jax.ShapeDtypeStruct((_N, 1), jnp.float32),
    )(x)
    return out.reshape(_N)


def _softlog_body(u_ref, o_ref):
    a = u_ref[...]
    amax = jnp.max(a)
    sumexp = jnp.abs(jnp.sum(jnp.exp(a - amax)))
    lse = jnp.log(sumexp) + amax
    w = jnp.exp(a - lse)
    o_ref[...] = jnp.log(w)


def _softlog(ulw):
    out = pl.pallas_call(
        _softlog_body,
        out_shape=jax.ShapeDtypeStruct((512, 128), jnp.float32),
    )(ulw.reshape(512, 128))
    return out.reshape(_N)


def _wrap(kd):
    return jax.random.wrap_key_data(jnp.array(kd, dtype=jnp.uint32),
                                    impl="threefry2x32")


def kernel(x0, w0):
    particles = x0
    logits = _log_w(w0)
    for t in range(_NT - 1):
        kns, kus, rk = _KEY_SCHED[t]
        noise = jnp.stack([jax.random.normal(_wrap(kn), (_N, _D), jnp.float32)
                           for kn in kns])
        us = jnp.stack([jax.random.uniform(_wrap(ku), (_N,), jnp.float32)
                        for ku in kus])
        cur = _mala(particles, noise, us, _MALA_CONSTS[t])
        pick = _categorical(rk, logits)
        resampled = _sc_gather(cur, pick)
        if t + 1 < _NT - 1:
            mu0 = np.float32(float(_MUS[t]))
            sig20 = np.float32(float(_SIGMAS[t]) ** 2)
            mu1 = np.float32(float(_MUS[t + 1]))
            sig21 = np.float32(float(_SIGMAS[t + 1]) ** 2)
            ulw = _ulw(resampled, (mu0, sig20, mu1, sig21, _LOGN))
            logits = _softlog(ulw)
        particles = resampled
    return particles[:, :_D]


# speedup vs baseline: 1.0134x; 1.0088x over previous
import functools

import numpy as np
import jax
import jax.numpy as jnp
from jax import lax
from jax.experimental import pallas as pl
from jax.experimental.pallas import tpu as pltpu
from jax.experimental.pallas import tpu_sc as plsc

_N = 65536
_D = 32
_NT = 3
_NK = 10

_MUS = np.linspace(-2.0, 2.0, _NT)
_SIGMAS = np.linspace(3.0, 1.0, _NT)

_M32 = 0xFFFFFFFF


def _tf2x32(k1, k2, c1, c2):
    ks0, ks1 = k1, k2
    ks2 = ks0 ^ ks1 ^ 0x1BD11BDA

    def rot(v, d):
        return ((v << d) | (v >> (32 - d))) & _M32

    def rounds(x0, x1, rs):
        for r in rs:
            x0 = (x0 + x1) & _M32
            x1 = rot(x1, r)
            x1 = x0 ^ x1
        return x0, x1

    x0 = (c1 + ks0) & _M32
    x1 = (c2 + ks1) & _M32
    x0, x1 = rounds(x0, x1, (13, 15, 26, 6))
    x0, x1 = (x0 + ks1) & _M32, (x1 + ks2 + 1) & _M32
    x0, x1 = rounds(x0, x1, (17, 29, 16, 24))
    x0, x1 = (x0 + ks2) & _M32, (x1 + ks0 + 2) & _M32
    x0, x1 = rounds(x0, x1, (13, 15, 26, 6))
    x0, x1 = (x0 + ks0) & _M32, (x1 + ks1 + 3) & _M32
    x0, x1 = rounds(x0, x1, (17, 29, 16, 24))
    x0, x1 = (x0 + ks1) & _M32, (x1 + ks2 + 4) & _M32
    x0, x1 = rounds(x0, x1, (13, 15, 26, 6))
    x0, x1 = (x0 + ks2) & _M32, (x1 + ks0 + 5) & _M32
    return x0, x1


def _split2(k):
    return _tf2x32(k[0], k[1], 0, 0), _tf2x32(k[0], k[1], 0, 1)


def _key_schedule():
    key = (0, 42)
    sched = []
    for _t in range(_NT - 1):
        kns, kus = [], []
        for _k in range(_NK):
            key, sk = _split2(key)
            kn, ku = _split2(sk)
            kns.append(kn)
            kus.append(ku)
        key, rk = _split2(key)
        sched.append((kns, kus, rk))
    return sched


_KEY_SCHED = _key_schedule()


def _mala_body(consts, x_ref, n_ref, u_ref, o_ref):
    x = x_ref[:, :_D]
    for k in range(_NK):
        tau, s2t, q2t, cq, ds, mu, sig2 = consts[k]
        noise = n_ref[k]
        u = u_ref[k][:, None]
        grad = np.float32(2.0) * (ds * (x - mu))
        new_x = x + tau * grad + s2t * noise

        def ld(y):
            dy = y - mu
            return (np.float32(-0.5) * jnp.sum(dy * dy, axis=-1, keepdims=True)) / sig2

        def log_q(xp, xx):
            mean = xx + tau * (np.float32(2.0) * (ds * (xx - mu)))
            dd = xp - mean
            return (np.float32(-0.5) * jnp.sum(dd * dd, axis=-1, keepdims=True)) / q2t - cq

        alp = ld(new_x) - ld(x) + log_q(x, new_x) - log_q(new_x, x)
        mask = (u < jnp.exp(alp)).astype(jnp.float32)
        x = mask * new_x + (np.float32(1.0) - mask) * x
    o_ref[:, :_D] = x


_MALA_BR = 2048
_W = 128


def _mala(x, noise, us, consts):
    grid = (_N // _MALA_BR,)
    win = x.shape[1]
    return pl.pallas_call(
        functools.partial(_mala_body, consts),
        grid=grid,
        in_specs=[
            pl.BlockSpec((_MALA_BR, win), lambda i: (i, 0)),
            pl.BlockSpec((_NK, _MALA_BR, _D), lambda i: (0, i, 0)),
            pl.BlockSpec((_NK, _MALA_BR), lambda i: (0, i)),
        ],
        out_specs=pl.BlockSpec((_MALA_BR, _W), lambda i: (i, 0)),
        out_shape=jax.ShapeDtypeStruct((_N, _W), jnp.float32),
    )(x, noise, us)


def _mala_consts(t):
    mu = np.float32(float(_MUS[t]))
    sig2 = np.float32(float(_SIGMAS[t]) ** 2)
    ds = np.float32(-0.5) * (np.float32(1.0) / sig2)
    out = []
    for k in range(1, _NK + 1):
        tau = 1.0 / k
        s2t = np.float32(float(jnp.sqrt(jnp.asarray(2.0 * tau, jnp.float32))))
        cq = np.float32(float(0.5 * _D * jnp.log(2.0 * jnp.pi * 2.0 * tau)))
        out.append((np.float32(tau), s2t, np.float32(2.0 * tau), cq, ds, mu, sig2))
    return tuple(out)


_MALA_CONSTS = (_mala_consts(0), _mala_consts(1))
_LOGN = np.float32(float(jnp.log(jnp.asarray(float(_N), jnp.float32))))
_TINY = np.float32(np.finfo(np.float32).tiny)
_MV = np.float32(np.float32(1.0) - _TINY)


_CAT_BR = 512
_CAT_BC = 2048


def _cat_body(rk, br, bc, ncols, l_ref, o_ref, bv_ref, bi_ref):
    i = pl.program_id(0)
    j = pl.program_id(1)
    nj = pl.num_programs(1)
    u32 = np.uint32

    rows = (lax.broadcasted_iota(jnp.int32, (br, bc), 0) + i * br).astype(jnp.uint32)
    cols = (lax.broadcasted_iota(jnp.int32, (br, bc), 1) + j * bc).astype(jnp.uint32)
    lo = rows * u32(ncols) + cols

    ks0, ks1 = rk
    ks2 = ks0 ^ ks1 ^ 0x1BD11BDA

    def rot(v, d):
        return (v << u32(d)) | (v >> u32(32 - d))

    def rounds(x0, x1, rs):
        for r in rs:
            x0 = x0 + x1
            x1 = rot(x1, r)
            x1 = x0 ^ x1
        return x0, x1

    x0 = jnp.full((br, bc), u32(ks0), jnp.uint32)
    x1 = lo + u32(ks1)
    x0, x1 = rounds(x0, x1, (13, 15, 26, 6))
    x0, x1 = x0 + u32(ks1), x1 + u32((ks2 + 1) & _M32)
    x0, x1 = rounds(x0, x1, (17, 29, 16, 24))
    x0, x1 = x0 + u32(ks2), x1 + u32((ks0 + 2) & _M32)
    x0, x1 = rounds(x0, x1, (13, 15, 26, 6))
    x0, x1 = x0 + u32(ks0), x1 + u32((ks1 + 3) & _M32)
    x0, x1 = rounds(x0, x1, (17, 29, 16, 24))
    x0, x1 = x0 + u32(ks1), x1 + u32((ks2 + 4) & _M32)
    x0, x1 = rounds(x0, x1, (13, 15, 26, 6))
    bits = (x0 + u32(ks2)) ^ (x1 + u32((ks0 + 5) & _M32))

    fb = (bits >> u32(9)) | u32(0x3F800000)
    f = lax.bitcast_convert_type(fb, jnp.float32) - np.float32(1.0)
    uu = jnp.maximum(_TINY, f * _MV + _TINY)
    g = -jnp.log(-jnp.log(uu))
    v = g + l_ref[...].reshape(1, bc)

    icols = lax.broadcasted_iota(jnp.int32, (br, bc), 1) + j * bc
    bmax = jnp.max(v, axis=1, keepdims=True)
    bidx = jnp.min(jnp.where(v == bmax, icols, jnp.int32(2147483647)),
                   axis=1, keepdims=True)

    @pl.when(j == 0)
    def _():
        bv_ref[...] = bmax
        bi_ref[...] = bidx

    @pl.when(j != 0)
    def _():
        better = bmax > bv_ref[...]
        bv_ref[...] = jnp.where(better, bmax, bv_ref[...])
        bi_ref[...] = jnp.where(better, bidx, bi_ref[...])

    @pl.when(j == nj - 1)
    def _():
        o_ref[...] = bi_ref[...].reshape(1, br, 1)


def _categorical(rk, logits):
    ni = _N // _CAT_BR
    nj = _N // _CAT_BC
    out = pl.pallas_call(
        functools.partial(_cat_body, rk, _CAT_BR, _CAT_BC, _N),
        grid=(ni, nj),
        in_specs=[
            pl.BlockSpec((1, 1, _CAT_BC), lambda i, j: (j, 0, 0)),
        ],
        out_specs=pl.BlockSpec((1, _CAT_BR, 1), lambda i, j: (i, 0, 0)),
        out_shape=jax.ShapeDtypeStruct((ni, _CAT_BR, 1), jnp.int32),
        scratch_shapes=[
            pltpu.VMEM((_CAT_BR, 1), jnp.float32),
            pltpu.VMEM((_CAT_BR, 1), jnp.int32),
        ],
    )(logits.reshape(nj, 1, _CAT_BC))
    return out.reshape(_N)


def _sc_gather(table, idx):
    info = plsc.get_sparse_core_info()
    nw = info.num_cores * info.num_subcores
    bpw = _N // nw
    chunk = 512
    mesh = plsc.VectorSubcoreMesh(core_axis_name="c", subcore_axis_name="s")

    @functools.partial(
        pl.kernel,
        mesh=mesh,
        out_type=jax.ShapeDtypeStruct((_N, _W), jnp.float32),
        scratch_types=[
            pltpu.VMEM((chunk,), jnp.int32),
            pltpu.VMEM((chunk, _W), jnp.float32),
            pltpu.SemaphoreType.DMA,
        ],
    )
    def gk(table_hbm, idx_hbm, out_hbm, idx_v, rows_v, sem):
        wid = lax.axis_index("s") * info.num_cores + lax.axis_index("c")
        base = wid * bpw
        for c in range(bpw // chunk):
            off = base + c * chunk
            pltpu.sync_copy(idx_hbm.at[pl.ds(off, chunk)], idx_v)
            pltpu.async_copy(table_hbm.at[idx_v], rows_v, sem).wait()
            pltpu.sync_copy(rows_v, out_hbm.at[pl.ds(off, chunk)])

    return gk(table, idx)


def _log_body(w_ref, o_ref):
    o_ref[...] = jnp.log(w_ref[...])


def _log_w(w):
    out = pl.pallas_call(
        _log_body,
        out_shape=jax.ShapeDtypeStruct((512, 128), jnp.float32),
    )(w.reshape(512, 128))
    return out.reshape(_N)


def _ulw_body(consts, x_ref, o_ref):
    mu0, sig20, mu1, sig21, logn = consts
    x = x_ref[:, :_D]

    def ld(y, mu, sig2):
        dy = y - mu
        return (np.float32(-0.5) * jnp.sum(dy * dy, axis=-1, keepdims=True)) / sig2

    o_ref[...] = ld(x, mu1, sig21) - logn - ld(x, mu0, sig20)


def _ulw(x, consts):
    grid = (_N // _MALA_BR,)
    out = pl.pallas_call(
        functools.partial(_ulw_body, consts),
        grid=grid,
        in_specs=[pl.BlockSpec((_MALA_BR, _W), lambda i: (i, 0))],
        out_specs=pl.BlockSpec((_MALA_BR, 1), lambda i: (i, 0)),
        out_shape=jax.ShapeDtypeStruct((_N, 1), jnp.float32),
    )(x)
    return out.reshape(_N)


def _softlog_body(u_ref, o_ref):
    a = u_ref[...]
    amax = jnp.max(a)
    sumexp = jnp.abs(jnp.sum(jnp.exp(a - amax)))
    lse = jnp.log(sumexp) + amax
    w = jnp.exp(a - lse)
    o_ref[...] = jnp.log(w)


def _softlog(ulw):
    out = pl.pallas_call(
        _softlog_body,
        out_shape=jax.ShapeDtypeStruct((512, 128), jnp.float32),
    )(ulw.reshape(512, 128))
    return out.reshape(_N)


def _wrap(kd):
    return jax.random.wrap_key_data(jnp.array(kd, dtype=jnp.uint32),
                                    impl="threefry2x32")


def kernel(x0, w0):
    particles = x0
    logits = _log_w(w0)
    for t in range(_NT - 1):
        kns, kus, rk = _KEY_SCHED[t]
        noise = jnp.stack([jax.random.normal(_wrap(kn), (_N, _D), jnp.float32)
                           for kn in kns])
        us = jnp.stack([jax.random.uniform(_wrap(ku), (_N,), jnp.float32)
                        for ku in kus])
        cur = _mala(particles, noise, us, _MALA_CONSTS[t])
        pick = _categorical(rk, logits)
        resampled = _sc_gather(cur, pick)
        if t + 1 < _NT - 1:
            mu0 = np.float32(float(_MUS[t]))
            sig20 = np.float32(float(_SIGMAS[t]) ** 2)
            mu1 = np.float32(float(_MUS[t + 1]))
            sig21 = np.float32(float(_SIGMAS[t + 1]) ** 2)
            ulw = _ulw(resampled, (mu0, sig20, mu1, sig21, _LOGN))
            logits = _softlog(ulw)
        particles = resampled
    return particles[:, :_D]
